# baseline (speedup 1.0000x reference)
import functools

import jax
import jax.numpy as jnp
from jax import lax
from jax.experimental import pallas as pl
from jax.experimental.pallas import tpu as pltpu
from jax.experimental.pallas import tpu_sc as plsc

_EULER_GAMMA = 0.5772156649015329
_EPS_VOL = 1e-23
_TINY = 1e-13
_CLAMP = 10.0

_B, _N, _D, _LE, _LN = 128, 32, 512, 64, 64
_NC, _NS, _NL = 2, 16, 16
_BPW = _B // (_NC * _NS)


_LOG2E = 1.4426950408889634
_LN2 = 0.6931471805599453


def _l1pe_mm(mn, mx):
    return jnp.log(1.0 + jnp.exp2(_LOG2E * (mn - mx)))


def _log2_vol_terms(x):
    sp = jnp.maximum(x, 0.0) + _l1pe_mm(jnp.minimum(x, 0.0), jnp.maximum(x, 0.0))
    return jnp.log2(sp + _EPS_VOL)


def _log_vol(l, r):
    return _LN2 * jnp.sum(_log2_vol_terms(r - l - 2.0 * _EULER_GAMMA), axis=-1)


def _pred_body(e_ref, nb_ref, out_ref):
    el = e_ref[:, 0, :]
    er = e_ref[:, 1, :]
    nl = nb_ref[:, :, 0, :]
    nr = nb_ref[:, :, 1, :]
    el_b = el[:, None, :]
    er_b = er[:, None, :]
    il = jnp.maximum(el_b, nl) + _l1pe_mm(jnp.minimum(el_b, nl), jnp.maximum(el_b, nl))
    ir = jnp.minimum(er_b, nr) - _l1pe_mm(jnp.minimum(er_b, nr), jnp.maximum(er_b, nr))
    log_int = _log_vol(il, ir)
    log_ent = _log_vol(el, er)[:, None]
    log_neigh = _log_vol(nl, nr)
    mx = jnp.maximum(log_ent, log_neigh)
    log_sum = mx + _l1pe_mm(jnp.minimum(log_ent, log_neigh), mx)
    d = jnp.minimum(log_int - log_sum, -1e-7)
    log_union = log_sum + jnp.log1p(-jnp.exp(d))
    log_pred = log_int - log_union
    out_ref[:, :] = jnp.exp(jnp.clip(log_pred, -_CLAMP, _CLAMP))


def _pred_pallas(entity_boxes, neighbor_boxes):
    bb = 8
    grid = (_B // bb,)
    return pl.pallas_call(
        _pred_body,
        grid=grid,
        in_specs=[
            pl.BlockSpec((bb, 2, _D), lambda i: (i, 0, 0)),
            pl.BlockSpec((bb, _N, 2, _D), lambda i: (i, 0, 0, 0)),
        ],
        out_specs=pl.BlockSpec((bb, _N), lambda i: (i, 0)),
        out_shape=jax.ShapeDtypeStruct((_B, _N), jnp.float32),
    )(entity_boxes, neighbor_boxes)


def _sc_target(ent_rels, ent_lens, neigh_rels, neigh_lens):
    ent_t = jnp.transpose(ent_rels, (1, 0))
    neigh_t = jnp.transpose(neigh_rels, (1, 2, 0))
    nlens_t = jnp.transpose(neigh_lens, (1, 0))
    mesh = plsc.VectorSubcoreMesh(core_axis_name="c", subcore_axis_name="s")

    @functools.partial(
        pl.kernel,
        out_type=(jax.ShapeDtypeStruct((_N, _B), jnp.float32),
                  jax.ShapeDtypeStruct((_N, _B), jnp.float32)),
        mesh=mesh,
        compiler_params=pltpu.CompilerParams(needs_layout_passes=False),
        scratch_types=[
            pltpu.VMEM((_LE, _B), jnp.int32),
            pltpu.VMEM((_B,), jnp.int32),
            pltpu.VMEM((_LN, _B), jnp.int32),
            pltpu.VMEM((_B,), jnp.int32),
            pltpu.VMEM((512,), jnp.int32),
            pltpu.VMEM((_B,), jnp.float32),
            pltpu.VMEM((_B,), jnp.float32),
        ],
    )
    def body(et_hbm, elen_hbm, nt_hbm, nlen_hbm, tgt_hbm, msk_hbm,
             ev, elv, nv, nlv, hist, tl, ml):
        wid = lax.axis_index("s") * _NC + lax.axis_index("c")
        pltpu.sync_copy(et_hbm, ev)
        pltpu.sync_copy(elen_hbm, elv)
        pltpu.sync_copy(nt_hbm.at[wid], nv)
        pltpu.sync_copy(nlen_hbm.at[wid], nlv)
        lanes = lax.broadcasted_iota(jnp.int32, (_NL,), 0)
        for k in range(512 // _NL):
            hist[pl.ds(k * _NL, _NL)] = jnp.zeros((_NL,), jnp.int32)
        for h in range(_B // _NL):
            def bbody(i, carry, h=h):
                t_out, m_out = carry
                b = h * _NL + i
                bf = jnp.full((_NL,), b, jnp.int32)
                le_vec = plsc.load_gather(elv, [bf])
                ln_vec = plsc.load_gather(nlv, [bf])
                vs, ms, cnts, lasts = [], [], [], []
                for k in range(_LE // _NL):
                    v = plsc.load_gather(ev, [lanes + k * _NL, bf])
                    m = (lanes + k * _NL) < le_vec
                    cnt, last = plsc.scan_count(v, m)
                    plsc.addupdate_scatter(hist, [v], cnt, mask=last)
                    vs.append(v); ms.append(m); cnts.append(cnt); lasts.append(last)
                for k in range(_LN // _NL):
                    v = plsc.load_gather(nv, [lanes + k * _NL, bf])
                    m = (lanes + k * _NL) < ln_vec
                    cnt, last = plsc.scan_count(v, m)
                    plsc.addupdate_scatter(hist, [v], cnt, mask=last)
                    vs.append(v); ms.append(m); cnts.append(cnt); lasts.append(last)
                acc = jnp.zeros((_NL,), jnp.float32)
                for v, m in zip(vs, ms):
                    occ = plsc.load_gather(hist, [v], mask=m)
                    occ_f = occ.astype(jnp.float32)
                    acc = acc + jnp.where(m & (occ >= 2), 1.0 / occ_f, 0.0)
                for v, cnt, last in zip(vs, cnts, lasts):
                    plsc.addupdate_scatter(hist, [v], -cnt, mask=last)
                inter = jnp.full((_NL,), jnp.sum(acc), jnp.float32)
                t_vec = inter / (le_vec.astype(jnp.float32)
                                 + ln_vec.astype(jnp.float32) + _TINY)
                m_vec = jnp.where((le_vec >= 1) & (ln_vec >= 1), 1.0, 0.0)
                sel = lanes == i
                return (jnp.where(sel, t_vec, t_out), jnp.where(sel, m_vec, m_out))
            z = jnp.zeros((_NL,), jnp.float32)
            t_out, m_out = lax.fori_loop(0, _NL, bbody, (z, z))
            tl[pl.ds(h * _NL, _NL)] = t_out
            ml[pl.ds(h * _NL, _NL)] = m_out
        pltpu.sync_copy(tl, tgt_hbm.at[wid])
        pltpu.sync_copy(ml, msk_hbm.at[wid])

    return body(ent_t, ent_lens, neigh_t, nlens_t)


def _combine_body(pred_ref, tgt_ref, msk_ref, out_ref):
    pred = jnp.transpose(pred_ref[...], (1, 0))
    tgt = tgt_ref[...]
    msk = msk_ref[...]
    loss = msk * (pred - tgt) ** 2
    out_ref[0, 0] = jnp.sum(loss) / float(_B * _N)


def _combine_pallas(pred, tgt_t, msk_t):
    return pl.pallas_call(
        _combine_body,
        in_specs=[
            pl.BlockSpec(memory_space=pltpu.VMEM),
            pl.BlockSpec(memory_space=pltpu.VMEM),
            pl.BlockSpec(memory_space=pltpu.VMEM),
        ],
        out_specs=pl.BlockSpec(memory_space=pltpu.SMEM),
        out_shape=jax.ShapeDtypeStruct((1, 1), jnp.float32),
    )(pred, tgt_t, msk_t)


def kernel(entity_boxes, neighbor_boxes, entity_relations, entity_rel_lens,
           neighbor_relations, neighbor_rel_lens):
    tgt_t, msk_t = _sc_target(entity_relations, entity_rel_lens,
                              neighbor_relations, neighbor_rel_lens)
    pred = _pred_pallas(entity_boxes, neighbor_boxes)
    out = _combine_pallas(pred, tgt_t, msk_t)
    return out[0, 0]


# speedup vs baseline: 1.0170x; 1.0170x over previous
import functools

import jax
import jax.numpy as jnp
from jax import lax
from jax.experimental import pallas as pl
from jax.experimental.pallas import tpu as pltpu
from jax.experimental.pallas import tpu_sc as plsc

_EULER_GAMMA = 0.5772156649015329
_EPS_VOL = 1e-23
_TINY = 1e-13
_CLAMP = 10.0

_B, _N, _D, _LE, _LN = 128, 32, 512, 64, 64
_NC, _NS, _NL = 2, 16, 16
_BPW = _B // (_NC * _NS)


_LOG2E = 1.4426950408889634
_LN2 = 0.6931471805599453


def _l1pe(x):
    return jnp.log(1.0 + jnp.exp(-jnp.abs(x)))


def _log2_vol_terms(x):
    sp = jnp.maximum(x, 0.0) + _l1pe(x)
    return jnp.log2(sp + _EPS_VOL)


def _log_vol(l, r):
    return _LN2 * jnp.sum(_log2_vol_terms(r - l - 2.0 * _EULER_GAMMA), axis=-1)


def _pred_body(e_ref, nb_ref, out_ref):
    el = e_ref[:, 0, :]
    er = e_ref[:, 1, :]
    nl = nb_ref[:, :, 0, :]
    nr = nb_ref[:, :, 1, :]
    el_b = el[:, None, :]
    er_b = er[:, None, :]
    il = jnp.maximum(el_b, nl) + _l1pe(el_b - nl)
    ir = jnp.minimum(er_b, nr) - _l1pe(er_b - nr)
    log_int = _log_vol(il, ir)
    log_ent = _log_vol(el, er)[:, None]
    log_neigh = _log_vol(nl, nr)
    log_sum = jnp.maximum(log_ent, log_neigh) + _l1pe(log_ent - log_neigh)
    d = jnp.minimum(log_int - log_sum, -1e-7)
    log_union = log_sum + jnp.log1p(-jnp.exp(d))
    log_pred = log_int - log_union
    out_ref[:, :] = jnp.exp(jnp.clip(log_pred, -_CLAMP, _CLAMP))


def _pred_pallas(entity_boxes, neighbor_boxes):
    bb = 8
    grid = (_B // bb,)
    return pl.pallas_call(
        _pred_body,
        grid=grid,
        in_specs=[
            pl.BlockSpec((bb, 2, _D), lambda i: (i, 0, 0)),
            pl.BlockSpec((bb, _N, 2, _D), lambda i: (i, 0, 0, 0)),
        ],
        out_specs=pl.BlockSpec((bb, _N), lambda i: (i, 0)),
        out_shape=jax.ShapeDtypeStruct((_B, _N), jnp.float32),
    )(entity_boxes, neighbor_boxes)


def _sc_target(ent_rels, ent_lens, neigh_rels, neigh_lens):
    mesh = plsc.VectorSubcoreMesh(core_axis_name="c", subcore_axis_name="s")

    @functools.partial(
        pl.kernel,
        out_type=(jax.ShapeDtypeStruct((_B, _N), jnp.float32),
                  jax.ShapeDtypeStruct((_B, _N), jnp.float32)),
        mesh=mesh,
        compiler_params=pltpu.CompilerParams(needs_layout_passes=False),
        scratch_types=[
            pltpu.VMEM((_BPW, _LE), jnp.int32),
            pltpu.VMEM((_B,), jnp.int32),
            pltpu.VMEM((_BPW, _N, _LN), jnp.int32),
            pltpu.VMEM((_BPW, _N), jnp.int32),
            pltpu.VMEM((512,), jnp.int32),
            pltpu.VMEM((_BPW, _N), jnp.float32),
            pltpu.VMEM((_BPW, _N), jnp.float32),
        ],
    )
    def body(er_hbm, elen_hbm, nr_hbm, nlen_hbm, tgt_hbm, msk_hbm,
             ev, elv, nv, nlv, hist, tl, ml):
        wid = lax.axis_index("s") * _NC + lax.axis_index("c")
        b0 = wid * _BPW
        pltpu.sync_copy(er_hbm.at[pl.ds(b0, _BPW)], ev)
        pltpu.sync_copy(elen_hbm, elv)
        pltpu.sync_copy(nr_hbm.at[pl.ds(b0, _BPW)], nv)
        pltpu.sync_copy(nlen_hbm.at[pl.ds(b0, _BPW)], nlv)
        lanes = lax.broadcasted_iota(jnp.int32, (_NL,), 0)
        for k in range(512 // _NL):
            hist[pl.ds(k * _NL, _NL)] = jnp.zeros((_NL,), jnp.int32)
        for j in range(_BPW):
            jf = jnp.full((_NL,), j, jnp.int32)
            le_vec = plsc.load_gather(elv, [jnp.full((_NL,), b0 + j, jnp.int32)])
            le_f = le_vec.astype(jnp.float32)
            evs, ems, ecnts, elasts = [], [], [], []
            for k in range(_LE // _NL):
                v = ev[j, pl.ds(k * _NL, _NL)]
                m = (lanes + k * _NL) < le_vec
                cnt, last = plsc.scan_count(v, m)
                plsc.addupdate_scatter(hist, [v], cnt, mask=last)
                evs.append(v); ems.append(m); ecnts.append(cnt); elasts.append(last)
            for h in range(2):
                def nbody(i, carry, h=h, jf=jf, le_vec=le_vec, le_f=le_f,
                          evs=evs, ems=ems):
                    t_out, m_out = carry
                    n = h * _NL + i
                    nf = jnp.full((_NL,), n, jnp.int32)
                    ln_vec = plsc.load_gather(nlv, [jf, nf])
                    nvs, nms, ncnts, nlasts = [], [], [], []
                    for k in range(_LN // _NL):
                        vv = plsc.load_gather(nv, [jf, nf, lanes + k * _NL])
                        mm = (lanes + k * _NL) < ln_vec
                        cnt, last = plsc.scan_count(vv, mm)
                        plsc.addupdate_scatter(hist, [vv], cnt, mask=last)
                        nvs.append(vv); nms.append(mm); ncnts.append(cnt); nlasts.append(last)
                    acc = jnp.zeros((_NL,), jnp.float32)
                    for v, m in zip(evs + nvs, ems + nms):
                        occ = plsc.load_gather(hist, [v], mask=m)
                        occ_f = occ.astype(jnp.float32)
                        acc = acc + jnp.where(m & (occ >= 2), 1.0 / occ_f, 0.0)
                    for k in range(_LN // _NL):
                        plsc.addupdate_scatter(hist, [nvs[k]], -ncnts[k], mask=nlasts[k])
                    inter = jnp.full((_NL,), jnp.sum(acc), jnp.float32)
                    t_vec = inter / (le_f + ln_vec.astype(jnp.float32) + _TINY)
                    m_vec = jnp.where((le_vec >= 1) & (ln_vec >= 1), 1.0, 0.0)
                    sel = lanes == i
                    return (jnp.where(sel, t_vec, t_out),
                            jnp.where(sel, m_vec, m_out))
                z = jnp.zeros((_NL,), jnp.float32)
                t_out, m_out = lax.fori_loop(0, _NL, nbody, (z, z))
                tl[j, pl.ds(h * _NL, _NL)] = t_out
                ml[j, pl.ds(h * _NL, _NL)] = m_out
            for k in range(_LE // _NL):
                plsc.addupdate_scatter(hist, [evs[k]], -ecnts[k], mask=elasts[k])
        pltpu.sync_copy(tl, tgt_hbm.at[pl.ds(b0, _BPW)])
        pltpu.sync_copy(ml, msk_hbm.at[pl.ds(b0, _BPW)])

    return body(ent_rels, ent_lens, neigh_rels, neigh_lens)


def _combine_body(pred_ref, tgt_ref, msk_ref, out_ref):
    pred = pred_ref[...]
    tgt = tgt_ref[...]
    msk = msk_ref[...]
    loss = msk * (pred - tgt) ** 2
    out_ref[0, 0] = jnp.sum(loss) / float(_B * _N)


def _combine_pallas(pred, tgt_t, msk_t):
    return pl.pallas_call(
        _combine_body,
        in_specs=[
            pl.BlockSpec(memory_space=pltpu.VMEM),
            pl.BlockSpec(memory_space=pltpu.VMEM),
            pl.BlockSpec(memory_space=pltpu.VMEM),
        ],
        out_specs=pl.BlockSpec(memory_space=pltpu.SMEM),
        out_shape=jax.ShapeDtypeStruct((1, 1), jnp.float32),
    )(pred, tgt_t, msk_t)


def kernel(entity_boxes, neighbor_boxes, entity_relations, entity_rel_lens,
           neighbor_relations, neighbor_rel_lens):
    tgt_t, msk_t = _sc_target(entity_relations, entity_rel_lens,
                              neighbor_relations, neighbor_rel_lens)
    pred = _pred_pallas(entity_boxes, neighbor_boxes)
    out = _combine_pallas(pred, tgt_t, msk_t)
    return out[0, 0]


# speedup vs baseline: 1.0762x; 1.0582x over previous
import functools

import jax
import jax.numpy as jnp
from jax import lax
from jax.experimental import pallas as pl
from jax.experimental.pallas import tpu as pltpu
from jax.experimental.pallas import tpu_sc as plsc

_EULER_GAMMA = 0.5772156649015329
_EPS_VOL = 1e-23
_TINY = 1e-13
_CLAMP = 10.0

_B, _N, _D, _LE, _LN = 128, 32, 512, 64, 64
_NC, _NS, _NL = 2, 16, 16
_BPW = _B // (_NC * _NS)


_LOG2E = 1.4426950408889634
_LN2 = 0.6931471805599453


def _l1pe(x):
    return jnp.log(1.0 + jnp.exp(-jnp.abs(x)))


def _log2_vol_terms(x):
    sp = jnp.maximum(x, 0.0) + _l1pe(x)
    return jnp.log2(sp + _EPS_VOL)


def _log_vol(l, r):
    return _LN2 * jnp.sum(_log2_vol_terms(r - l - 2.0 * _EULER_GAMMA), axis=-1)


def _pred_body(e_ref, nb_ref, out_ref):
    el = e_ref[:, 0, :]
    er = e_ref[:, 1, :]
    nl = nb_ref[:, :, 0, :]
    nr = nb_ref[:, :, 1, :]
    el_b = el[:, None, :]
    er_b = er[:, None, :]
    il = jnp.maximum(el_b, nl) + _l1pe(el_b - nl)
    ir = jnp.minimum(er_b, nr) - _l1pe(er_b - nr)
    log_int = _log_vol(il, ir)
    log_ent = _log_vol(el, er)[:, None]
    log_neigh = _log_vol(nl, nr)
    log_sum = jnp.maximum(log_ent, log_neigh) + _l1pe(log_ent - log_neigh)
    d = jnp.minimum(log_int - log_sum, -1e-7)
    log_union = log_sum + jnp.log1p(-jnp.exp(d))
    log_pred = log_int - log_union
    out_ref[:, :] = jnp.exp(jnp.clip(log_pred, -_CLAMP, _CLAMP))


def _pred_pallas(entity_boxes, neighbor_boxes):
    bb = 16
    grid = (_B // bb,)
    return pl.pallas_call(
        _pred_body,
        grid=grid,
        in_specs=[
            pl.BlockSpec((bb, 2, _D), lambda i: (i, 0, 0)),
            pl.BlockSpec((bb, _N, 2, _D), lambda i: (i, 0, 0, 0)),
        ],
        out_specs=pl.BlockSpec((bb, _N), lambda i: (i, 0)),
        out_shape=jax.ShapeDtypeStruct((_B, _N), jnp.float32),
    )(entity_boxes, neighbor_boxes)


def _sc_target(ent_rels, ent_lens, neigh_rels, neigh_lens):
    mesh = plsc.VectorSubcoreMesh(core_axis_name="c", subcore_axis_name="s")

    @functools.partial(
        pl.kernel,
        out_type=(jax.ShapeDtypeStruct((_B, _N), jnp.float32),
                  jax.ShapeDtypeStruct((_B, _N), jnp.float32)),
        mesh=mesh,
        compiler_params=pltpu.CompilerParams(needs_layout_passes=False),
        scratch_types=[
            pltpu.VMEM((_BPW, _LE), jnp.int32),
            pltpu.VMEM((_B,), jnp.int32),
            pltpu.VMEM((_BPW, _N, _LN), jnp.int32),
            pltpu.VMEM((_BPW, _N), jnp.int32),
            pltpu.VMEM((512,), jnp.int32),
            pltpu.VMEM((_BPW, _N), jnp.float32),
            pltpu.VMEM((_BPW, _N), jnp.float32),
        ],
    )
    def body(er_hbm, elen_hbm, nr_hbm, nlen_hbm, tgt_hbm, msk_hbm,
             ev, elv, nv, nlv, hist, tl, ml):
        wid = lax.axis_index("s") * _NC + lax.axis_index("c")
        b0 = wid * _BPW
        pltpu.sync_copy(er_hbm.at[pl.ds(b0, _BPW)], ev)
        pltpu.sync_copy(elen_hbm, elv)
        pltpu.sync_copy(nr_hbm.at[pl.ds(b0, _BPW)], nv)
        pltpu.sync_copy(nlen_hbm.at[pl.ds(b0, _BPW)], nlv)
        lanes = lax.broadcasted_iota(jnp.int32, (_NL,), 0)
        for k in range(512 // _NL):
            hist[pl.ds(k * _NL, _NL)] = jnp.zeros((_NL,), jnp.int32)
        for j in range(_BPW):
            jf = jnp.full((_NL,), j, jnp.int32)
            le_vec = plsc.load_gather(elv, [jnp.full((_NL,), b0 + j, jnp.int32)])
            le_f = le_vec.astype(jnp.float32)
            evs, ems, ecnts, elasts = [], [], [], []
            for k in range(_LE // _NL):
                v = ev[j, pl.ds(k * _NL, _NL)]
                m = (lanes + k * _NL) < le_vec
                cnt, last = plsc.scan_count(v, m)
                plsc.addupdate_scatter(hist, [v], cnt, mask=last)
                evs.append(v); ems.append(m); ecnts.append(cnt); elasts.append(last)
            for h in range(2):
                def nbody(i, carry, h=h, jf=jf, le_vec=le_vec, le_f=le_f,
                          evs=evs, ems=ems):
                    t_out, m_out = carry
                    n = h * _NL + i
                    nf = jnp.full((_NL,), n, jnp.int32)
                    ln_vec = plsc.load_gather(nlv, [jf, nf])
                    nvs, nms, ncnts, nlasts = [], [], [], []
                    for k in range(_LN // _NL):
                        vv = plsc.load_gather(nv, [jf, nf, lanes + k * _NL])
                        mm = (lanes + k * _NL) < ln_vec
                        cnt, last = plsc.scan_count(vv, mm)
                        plsc.addupdate_scatter(hist, [vv], cnt, mask=last)
                        nvs.append(vv); nms.append(mm); ncnts.append(cnt); nlasts.append(last)
                    acc = jnp.zeros((_NL,), jnp.float32)
                    for v, m in zip(evs + nvs, ems + nms):
                        occ = plsc.load_gather(hist, [v], mask=m)
                        occ_f = occ.astype(jnp.float32)
                        acc = acc + jnp.where(m & (occ >= 2), 1.0 / occ_f, 0.0)
                    for k in range(_LN // _NL):
                        plsc.addupdate_scatter(hist, [nvs[k]], -ncnts[k], mask=nlasts[k])
                    inter = jnp.full((_NL,), jnp.sum(acc), jnp.float32)
                    t_vec = inter / (le_f + ln_vec.astype(jnp.float32) + _TINY)
                    m_vec = jnp.where((le_vec >= 1) & (ln_vec >= 1), 1.0, 0.0)
                    sel = lanes == i
                    return (jnp.where(sel, t_vec, t_out),
                            jnp.where(sel, m_vec, m_out))
                z = jnp.zeros((_NL,), jnp.float32)
                t_out, m_out = lax.fori_loop(0, _NL, nbody, (z, z))
                tl[j, pl.ds(h * _NL, _NL)] = t_out
                ml[j, pl.ds(h * _NL, _NL)] = m_out
            for k in range(_LE // _NL):
                plsc.addupdate_scatter(hist, [evs[k]], -ecnts[k], mask=elasts[k])
        pltpu.sync_copy(tl, tgt_hbm.at[pl.ds(b0, _BPW)])
        pltpu.sync_copy(ml, msk_hbm.at[pl.ds(b0, _BPW)])

    return body(ent_rels, ent_lens, neigh_rels, neigh_lens)


def _combine_body(pred_ref, tgt_ref, msk_ref, out_ref):
    pred = pred_ref[...]
    tgt = tgt_ref[...]
    msk = msk_ref[...]
    loss = msk * (pred - tgt) ** 2
    out_ref[0, 0] = jnp.sum(loss) / float(_B * _N)


def _combine_pallas(pred, tgt_t, msk_t):
    return pl.pallas_call(
        _combine_body,
        in_specs=[
            pl.BlockSpec(memory_space=pltpu.VMEM),
            pl.BlockSpec(memory_space=pltpu.VMEM),
            pl.BlockSpec(memory_space=pltpu.VMEM),
        ],
        out_specs=pl.BlockSpec(memory_space=pltpu.SMEM),
        out_shape=jax.ShapeDtypeStruct((1, 1), jnp.float32),
    )(pred, tgt_t, msk_t)


def kernel(entity_boxes, neighbor_boxes, entity_relations, entity_rel_lens,
           neighbor_relations, neighbor_rel_lens):
    tgt_t, msk_t = _sc_target(entity_relations, entity_rel_lens,
                              neighbor_relations, neighbor_rel_lens)
    pred = _pred_pallas(entity_boxes, neighbor_boxes)
    out = _combine_pallas(pred, tgt_t, msk_t)
    return out[0, 0]


# speedup vs baseline: 1.0882x; 1.0112x over previous
import functools

import jax
import jax.numpy as jnp
from jax import lax
from jax.experimental import pallas as pl
from jax.experimental.pallas import tpu as pltpu
from jax.experimental.pallas import tpu_sc as plsc

_EULER_GAMMA = 0.5772156649015329
_EPS_VOL = 1e-23
_TINY = 1e-13
_CLAMP = 10.0

_B, _N, _D, _LE, _LN = 128, 32, 512, 64, 64
_NC, _NS, _NL = 2, 16, 16
_BPW = _B // (_NC * _NS)


_LOG2E = 1.4426950408889634
_LN2 = 0.6931471805599453


def _l1pe(x):
    return jnp.log(1.0 + jnp.exp(-jnp.abs(x)))


def _log2_vol_terms(x):
    sp = jnp.maximum(x, 0.0) + _l1pe(x)
    return jnp.log2(sp + _EPS_VOL)


def _log_vol(l, r):
    return _LN2 * jnp.sum(_log2_vol_terms(r - l - 2.0 * _EULER_GAMMA), axis=-1)


def _pred_body(e_ref, nb_ref, out_ref):
    el = e_ref[:, 0, :]
    er = e_ref[:, 1, :]
    nl = nb_ref[:, :, 0, :]
    nr = nb_ref[:, :, 1, :]
    el_b = el[:, None, :]
    er_b = er[:, None, :]
    il = jnp.maximum(el_b, nl) + _l1pe(el_b - nl)
    ir = jnp.minimum(er_b, nr) - _l1pe(er_b - nr)
    log_int = _log_vol(il, ir)
    log_ent = _log_vol(el, er)[:, None]
    log_neigh = _log_vol(nl, nr)
    log_sum = jnp.maximum(log_ent, log_neigh) + _l1pe(log_ent - log_neigh)
    d = jnp.minimum(log_int - log_sum, -1e-7)
    log_union = log_sum + jnp.log1p(-jnp.exp(d))
    log_pred = log_int - log_union
    out_ref[:, :] = jnp.exp(jnp.clip(log_pred, -_CLAMP, _CLAMP))


def _pred_pallas(entity_boxes, neighbor_boxes):
    bb = 32
    grid = (_B // bb,)
    return pl.pallas_call(
        _pred_body,
        grid=grid,
        in_specs=[
            pl.BlockSpec((bb, 2, _D), lambda i: (i, 0, 0)),
            pl.BlockSpec((bb, _N, 2, _D), lambda i: (i, 0, 0, 0)),
        ],
        out_specs=pl.BlockSpec((bb, _N), lambda i: (i, 0)),
        out_shape=jax.ShapeDtypeStruct((_B, _N), jnp.float32),
    )(entity_boxes, neighbor_boxes)


def _sc_target(ent_rels, ent_lens, neigh_rels, neigh_lens):
    mesh = plsc.VectorSubcoreMesh(core_axis_name="c", subcore_axis_name="s")

    @functools.partial(
        pl.kernel,
        out_type=(jax.ShapeDtypeStruct((_B, _N), jnp.float32),
                  jax.ShapeDtypeStruct((_B, _N), jnp.float32)),
        mesh=mesh,
        compiler_params=pltpu.CompilerParams(needs_layout_passes=False),
        scratch_types=[
            pltpu.VMEM((_BPW, _LE), jnp.int32),
            pltpu.VMEM((_B,), jnp.int32),
            pltpu.VMEM((_BPW, _N, _LN), jnp.int32),
            pltpu.VMEM((_BPW, _N), jnp.int32),
            pltpu.VMEM((512,), jnp.int32),
            pltpu.VMEM((_BPW, _N), jnp.float32),
            pltpu.VMEM((_BPW, _N), jnp.float32),
        ],
    )
    def body(er_hbm, elen_hbm, nr_hbm, nlen_hbm, tgt_hbm, msk_hbm,
             ev, elv, nv, nlv, hist, tl, ml):
        wid = lax.axis_index("s") * _NC + lax.axis_index("c")
        b0 = wid * _BPW
        pltpu.sync_copy(er_hbm.at[pl.ds(b0, _BPW)], ev)
        pltpu.sync_copy(elen_hbm, elv)
        pltpu.sync_copy(nr_hbm.at[pl.ds(b0, _BPW)], nv)
        pltpu.sync_copy(nlen_hbm.at[pl.ds(b0, _BPW)], nlv)
        lanes = lax.broadcasted_iota(jnp.int32, (_NL,), 0)
        for k in range(512 // _NL):
            hist[pl.ds(k * _NL, _NL)] = jnp.zeros((_NL,), jnp.int32)
        for j in range(_BPW):
            jf = jnp.full((_NL,), j, jnp.int32)
            le_vec = plsc.load_gather(elv, [jnp.full((_NL,), b0 + j, jnp.int32)])
            le_f = le_vec.astype(jnp.float32)
            evs, ems, ecnts, elasts = [], [], [], []
            for k in range(_LE // _NL):
                v = ev[j, pl.ds(k * _NL, _NL)]
                m = (lanes + k * _NL) < le_vec
                cnt, last = plsc.scan_count(v, m)
                plsc.addupdate_scatter(hist, [v], cnt, mask=last)
                evs.append(v); ems.append(m); ecnts.append(cnt); elasts.append(last)
            for h in range(2):
                def nbody(i, carry, h=h, jf=jf, le_vec=le_vec, le_f=le_f,
                          evs=evs, ems=ems):
                    t_out, m_out = carry
                    n = h * _NL + i
                    nf = jnp.full((_NL,), n, jnp.int32)
                    ln_vec = plsc.load_gather(nlv, [jf, nf])
                    nvs, nms, ncnts, nlasts = [], [], [], []
                    for k in range(_LN // _NL):
                        vv = plsc.load_gather(nv, [jf, nf, lanes + k * _NL])
                        mm = (lanes + k * _NL) < ln_vec
                        cnt, last = plsc.scan_count(vv, mm)
                        plsc.addupdate_scatter(hist, [vv], cnt, mask=last)
                        nvs.append(vv); nms.append(mm); ncnts.append(cnt); nlasts.append(last)
                    acc = jnp.zeros((_NL,), jnp.float32)
                    for v, m in zip(evs + nvs, ems + nms):
                        occ = plsc.load_gather(hist, [v], mask=m)
                        occ_f = occ.astype(jnp.float32)
                        acc = acc + jnp.where(m & (occ >= 2), 1.0 / occ_f, 0.0)
                    for k in range(_LN // _NL):
                        plsc.addupdate_scatter(hist, [nvs[k]], -ncnts[k], mask=nlasts[k])
                    inter = jnp.full((_NL,), jnp.sum(acc), jnp.float32)
                    t_vec = inter / (le_f + ln_vec.astype(jnp.float32) + _TINY)
                    m_vec = jnp.where((le_vec >= 1) & (ln_vec >= 1), 1.0, 0.0)
                    sel = lanes == i
                    return (jnp.where(sel, t_vec, t_out),
                            jnp.where(sel, m_vec, m_out))
                z = jnp.zeros((_NL,), jnp.float32)
                t_out, m_out = lax.fori_loop(0, _NL, nbody, (z, z))
                tl[j, pl.ds(h * _NL, _NL)] = t_out
                ml[j, pl.ds(h * _NL, _NL)] = m_out
            for k in range(_LE // _NL):
                plsc.addupdate_scatter(hist, [evs[k]], -ecnts[k], mask=elasts[k])
        pltpu.sync_copy(tl, tgt_hbm.at[pl.ds(b0, _BPW)])
        pltpu.sync_copy(ml, msk_hbm.at[pl.ds(b0, _BPW)])

    return body(ent_rels, ent_lens, neigh_rels, neigh_lens)


def _combine_body(pred_ref, tgt_ref, msk_ref, out_ref):
    pred = pred_ref[...]
    tgt = tgt_ref[...]
    msk = msk_ref[...]
    loss = msk * (pred - tgt) ** 2
    out_ref[0, 0] = jnp.sum(loss) / float(_B * _N)


def _combine_pallas(pred, tgt_t, msk_t):
    return pl.pallas_call(
        _combine_body,
        in_specs=[
            pl.BlockSpec(memory_space=pltpu.VMEM),
            pl.BlockSpec(memory_space=pltpu.VMEM),
            pl.BlockSpec(memory_space=pltpu.VMEM),
        ],
        out_specs=pl.BlockSpec(memory_space=pltpu.SMEM),
        out_shape=jax.ShapeDtypeStruct((1, 1), jnp.float32),
    )(pred, tgt_t, msk_t)


def kernel(entity_boxes, neighbor_boxes, entity_relations, entity_rel_lens,
           neighbor_relations, neighbor_rel_lens):
    tgt_t, msk_t = _sc_target(entity_relations, entity_rel_lens,
                              neighbor_relations, neighbor_rel_lens)
    pred = _pred_pallas(entity_boxes, neighbor_boxes)
    out = _combine_pallas(pred, tgt_t, msk_t)
    return out[0, 0]
